# Initial kernel scaffold; baseline (speedup 1.0000x reference)
import jax
import jax.numpy as jnp
from jax.experimental import pallas as pl


def kernel(z, codebook):
    raise NotImplementedError("write your pallas kernel here")


# speedup vs baseline: 1.2171x; 1.2171x over previous
import functools

import jax
import jax.numpy as jnp
from jax import lax
from jax.experimental import pallas as pl
from jax.experimental.pallas import tpu as pltpu
from jax.experimental.pallas import tpu_sc as plsc

N_E = 8192
DIM = 256
BETA = 0.25


_BM = 256


def _argmin_body(z_blk, zsq_blk, cbt_blk, esq_blk, idx_ref, loss_ref, acc_ref):
    i = pl.program_id(0)
    n_steps = pl.num_programs(0)

    mm = jax.lax.dot_general(
        z_blk[...], cbt_blk[...],
        dimension_numbers=(((1,), (0,)), ((), ())),
        preferred_element_type=jnp.float32,
    )
    d2 = (zsq_blk[...] - 2.0 * mm) + esq_blk[...]
    m = jnp.min(d2, axis=1, keepdims=True)
    col = jax.lax.broadcasted_iota(jnp.int32, d2.shape, 1)
    idx = jnp.min(jnp.where(d2 == m, col, N_E), axis=1, keepdims=True)
    idx_ref[...] = idx

    blk_sum = jnp.sum(m)

    @pl.when(i == 0)
    def _():
        acc_ref[0] = blk_sum

    @pl.when(i > 0)
    def _():
        acc_ref[0] = acc_ref[0] + blk_sum

    @pl.when(i == n_steps - 1)
    def _():
        loss_ref[...] = jnp.full((1, 1), acc_ref[0], jnp.float32)


def _distances_argmin(z_flat, z_sq, cb_t, e_sq):
    n = z_flat.shape[0]
    grid = n // _BM
    idx, d2_sum = pl.pallas_call(
        _argmin_body,
        grid=(grid,),
        in_specs=[
            pl.BlockSpec((_BM, DIM), lambda i: (i, 0)),
            pl.BlockSpec((_BM, 1), lambda i: (i, 0)),
            pl.BlockSpec((DIM, N_E), lambda i: (0, 0)),
            pl.BlockSpec((1, N_E), lambda i: (0, 0)),
        ],
        out_specs=[
            pl.BlockSpec((_BM, 1), lambda i: (i, 0)),
            pl.BlockSpec((1, 1), lambda i: (0, 0)),
        ],
        out_shape=[
            jax.ShapeDtypeStruct((n, 1), jnp.int32),
            jax.ShapeDtypeStruct((1, 1), jnp.float32),
        ],
        scratch_shapes=[pltpu.SMEM((1,), jnp.float32)],
    )(z_flat, z_sq, cb_t, e_sq)
    return idx.reshape(n), d2_sum[0, 0]


_CHUNK = 128


def _make_gather(n_rows):
    info = plsc.get_sparse_core_info()
    nw = info.num_cores * info.num_subcores
    rows_per_w = n_rows // nw
    n_chunks = rows_per_w // _CHUNK
    mesh = plsc.VectorSubcoreMesh(core_axis_name="c", subcore_axis_name="s")

    @functools.partial(
        pl.kernel,
        mesh=mesh,
        out_type=jax.ShapeDtypeStruct((n_rows, DIM), jnp.float32),
        scratch_types=[
            pltpu.VMEM((_CHUNK,), jnp.int32),
            pltpu.VMEM((_CHUNK, DIM), jnp.float32),
            pltpu.SemaphoreType.DMA,
        ],
    )
    def gather(table_hbm, idx_hbm, out_hbm, idx_v, rows_v, sem):
        wid = lax.axis_index("s") * info.num_cores + lax.axis_index("c")
        base = wid * rows_per_w
        for c in range(n_chunks):
            off = base + c * _CHUNK
            pltpu.sync_copy(idx_hbm.at[pl.ds(off, _CHUNK)], idx_v)
            pltpu.async_copy(table_hbm.at[idx_v], rows_v, sem).wait()
            pltpu.sync_copy(rows_v, out_hbm.at[pl.ds(off, _CHUNK)])

    return gather


def kernel(z, codebook):
    zf = z.reshape(-1, z.shape[-1])
    n = zf.shape[0]
    z_sq = jnp.sum(zf * zf, axis=1, keepdims=True)
    e_sq = jnp.sum(codebook * codebook, axis=1)[None, :]
    cb_t = codebook.T

    indices, d2_sum = _distances_argmin(zf, z_sq, cb_t, e_sq)

    z_q = _make_gather(n)(codebook, indices).reshape(z.shape)

    loss = (1.0 + BETA) * d2_sum / (n * DIM)
    z_q_st = z + jax.lax.stop_gradient(z_q - z)
    return z_q_st, loss, indices
